# Initial kernel scaffold; baseline (speedup 1.0000x reference)
import functools

import jax
import jax.numpy as jnp
from jax import lax
from jax.experimental import pallas as pl
from jax.experimental.pallas import tpu as pltpu
from jax.experimental.pallas import tpu_sc as plsc

D = 32
NC = 2
NS = 16
NW = NC * NS
CHUNK = 1024


@functools.partial(jax.jit, static_argnums=(2,))
def _gather(table, idx_flat, n_total):
    b_per_w = n_total // NW
    n_chunks = b_per_w // CHUNK
    mesh = plsc.VectorSubcoreMesh(core_axis_name="c", subcore_axis_name="s")

    @functools.partial(
        pl.kernel,
        mesh=mesh,
        out_type=jax.ShapeDtypeStruct((n_total, D), jnp.float32),
        scratch_types=[
            pltpu.VMEM((CHUNK,), jnp.int32),
            pltpu.VMEM((CHUNK, D), jnp.float32),
            pltpu.SemaphoreType.DMA,
        ],
    )
    def k(table_hbm, idx_hbm, out_hbm, idx_v, rows_v, sem):
        wid = lax.axis_index("s") * NC + lax.axis_index("c")
        base = wid * b_per_w

        def body(g, carry):
            off = base + g * CHUNK
            pltpu.sync_copy(idx_hbm.at[pl.ds(off, CHUNK)], idx_v)
            pltpu.async_copy(table_hbm.at[idx_v], rows_v, sem).wait()
            pltpu.sync_copy(rows_v, out_hbm.at[pl.ds(off, CHUNK)])
            return carry

        lax.fori_loop(0, n_chunks, body, 0)

    return k(table, idx_flat)


def kernel(x, table):
    b, h = x.shape
    out = _gather(table, x.reshape(b * h), b * h)
    return out.reshape(b, h, D)


# speedup vs baseline: 1.0940x; 1.0940x over previous
import functools

import jax
import jax.numpy as jnp
from jax import lax
from jax.experimental import pallas as pl
from jax.experimental.pallas import tpu as pltpu
from jax.experimental.pallas import tpu_sc as plsc

D = 32
NC = 2
NS = 16
NW = NC * NS
CHUNK = 1024


@functools.partial(jax.jit, static_argnums=(2,))
def _gather(table, idx_flat, n_total):
    b_per_w = n_total // NW
    n_chunks = b_per_w // CHUNK
    mesh = plsc.VectorSubcoreMesh(core_axis_name="c", subcore_axis_name="s")

    @functools.partial(
        pl.kernel,
        mesh=mesh,
        out_type=jax.ShapeDtypeStruct((n_total, D), jnp.float32),
        scratch_types=[
            pltpu.VMEM((CHUNK,), jnp.int32),
            pltpu.VMEM((CHUNK, D), jnp.float32),
            pltpu.SemaphoreType.DMA,
        ],
        compiler_params=pltpu.CompilerParams(use_tc_tiling_on_sc=False),
    )
    def k(table_hbm, idx_hbm, out_hbm, idx_v, rows_v, sem):
        wid = lax.axis_index("s") * NC + lax.axis_index("c")
        base = wid * b_per_w

        def body(g, carry):
            off = base + g * CHUNK
            pltpu.sync_copy(idx_hbm.at[pl.ds(off, CHUNK)], idx_v)
            pltpu.async_copy(table_hbm.at[idx_v], rows_v, sem).wait()
            pltpu.sync_copy(rows_v, out_hbm.at[pl.ds(off, CHUNK)])
            return carry

        lax.fori_loop(0, n_chunks, body, 0)

    return k(table, idx_flat)


def kernel(x, table):
    b, h = x.shape
    out = _gather(table, x.reshape(b * h), b * h)
    return out.reshape(b, h, D)


# speedup vs baseline: 1.1128x; 1.0172x over previous
import functools

import jax
import jax.numpy as jnp
from jax import lax
from jax.experimental import pallas as pl
from jax.experimental.pallas import tpu as pltpu
from jax.experimental.pallas import tpu_sc as plsc

D = 32
NC = 2
NS = 16
NW = NC * NS
CHUNK = 640
NBUF = 4


@functools.partial(jax.jit, static_argnums=(2,))
def _gather(table, idx, n_total):
    b_per_w = n_total // NW
    n_chunks = b_per_w // CHUNK
    assert n_chunks % NBUF == 0 and n_chunks >= 2 * NBUF
    mesh = plsc.VectorSubcoreMesh(core_axis_name="c", subcore_axis_name="s")

    @functools.partial(
        pl.kernel,
        mesh=mesh,
        out_type=jax.ShapeDtypeStruct((n_total, D), jnp.float32),
        scratch_types=[
            [pltpu.VMEM((CHUNK,), jnp.int32) for _ in range(NBUF)],
            [pltpu.VMEM((CHUNK, D), jnp.float32) for _ in range(NBUF)],
            pltpu.SemaphoreType.DMA((NBUF,)),
            pltpu.SemaphoreType.DMA((NBUF,)),
            pltpu.SemaphoreType.DMA((NBUF,)),
        ],
        compiler_params=pltpu.CompilerParams(use_tc_tiling_on_sc=False),
    )
    def k(table_hbm, idx_hbm, out_hbm, ibufs, rows_v, sem_i, sem_g, sem_o):
        wid = lax.axis_index("s") * NC + lax.axis_index("c")
        base = wid * b_per_w

        def idx_copy(g, j):
            pltpu.make_async_copy(
                idx_hbm.at[pl.ds(base + g * CHUNK, CHUNK)], ibufs[j],
                sem_i.at[j],
            ).start()

        def wait_idx(j):
            pltpu.make_async_copy(
                idx_hbm.at[pl.ds(base, CHUNK)], ibufs[j], sem_i.at[j]
            ).wait()

        def fire(g, j):
            pltpu.make_async_copy(
                table_hbm.at[ibufs[j]], rows_v[j], sem_g.at[j]
            ).start()

        def wait_fire(j):
            pltpu.make_async_copy(
                table_hbm.at[ibufs[j]], rows_v[j], sem_g.at[j]
            ).wait()

        def store(g, j):
            pltpu.make_async_copy(
                rows_v[j], out_hbm.at[pl.ds(base + g * CHUNK, CHUNK)],
                sem_o.at[j],
            ).start()

        def wait_store(j):
            pltpu.make_async_copy(
                rows_v[j], out_hbm.at[pl.ds(base, CHUNK)], sem_o.at[j]
            ).wait()

        def step(g, j, first_round, last_a):
            wait_idx(j)
            if not first_round:
                wait_store(j)
            fire(g, j)
            p = (j + NBUF - 2) % NBUF
            wait_fire(p)
            store(g - 2, p)
            if not last_a:
                idx_copy(g + 2, p)

        for j in range(NBUF):
            idx_copy(j, j)
        for j in range(2):
            wait_idx(j)
            fire(j, j)
        for j in range(2, NBUF):
            wait_idx(j)
            fire(j, j)
            p = j - 2
            wait_fire(p)
            store(p, p)
            idx_copy(p + NBUF, p)

        def body(i, carry):
            g0 = i * NBUF
            for j in range(NBUF):
                step(g0 + j, j, False, False)
            return carry

        lax.fori_loop(1, n_chunks // NBUF - 1, body, 0)

        g0 = n_chunks - NBUF
        for j in range(NBUF):
            step(g0 + j, j, False, j >= 2)
        for j in range(2):
            p = (j + NBUF - 2) % NBUF
            wait_fire(p)
            store(n_chunks - 2 + j, p)
        for j in range(NBUF):
            wait_store(j)

    return k(table, idx)


def kernel(x, table):
    b, h = x.shape
    out = _gather(table, x.reshape(b * h), b * h)
    return out.reshape(b, h, D)


# speedup vs baseline: 1.4236x; 1.2793x over previous
import functools

import jax
import jax.numpy as jnp
from jax import lax
from jax.experimental import pallas as pl
from jax.experimental.pallas import tpu as pltpu
from jax.experimental.pallas import tpu_sc as plsc

D = 32
NC = 2
NS = 16
NW = NC * NS
H = 50
B = 16384
BB = 128
NBB = B // BB
NU = H * NBB
U_PER_W = NU // NW


@jax.jit
def _embed(table, x_t):
    mesh = plsc.VectorSubcoreMesh(core_axis_name="c", subcore_axis_name="s")

    @functools.partial(
        pl.kernel,
        mesh=mesh,
        out_type=jax.ShapeDtypeStruct((H, D // 8, NBB, 8, BB), jnp.float32),
        scratch_types=[
            [pltpu.VMEM((BB,), jnp.int32) for _ in range(2)],
            [pltpu.VMEM((BB, D), jnp.float32) for _ in range(2)],
            [pltpu.VMEM((D, BB), jnp.float32) for _ in range(2)],
            pltpu.SemaphoreType.DMA((2,)),
            pltpu.SemaphoreType.DMA((2,)),
            pltpu.SemaphoreType.DMA((2,)),
        ],
        compiler_params=pltpu.CompilerParams(
            use_tc_tiling_on_sc=False, needs_layout_passes=False
        ),
    )
    def k(table_hbm, x_hbm, out_hbm, ibuf, rows, tbuf, sem_i, sem_g, sem_o):
        wid = lax.axis_index("s") * NC + lax.axis_index("c")
        u0 = wid * U_PER_W

        def fire_idx(u, s):
            h, bb = u // NBB, u % NBB
            pltpu.make_async_copy(
                x_hbm.at[h, pl.ds(bb * BB, BB)], ibuf[s], sem_i.at[s]
            ).start()

        def wait_idx(s):
            pltpu.make_async_copy(
                x_hbm.at[0, pl.ds(0, BB)], ibuf[s], sem_i.at[s]
            ).wait()

        def fire_g(s):
            pltpu.make_async_copy(
                table_hbm.at[ibuf[s]], rows[s], sem_g.at[s]
            ).start()

        def wait_g(s):
            pltpu.make_async_copy(
                table_hbm.at[ibuf[s]], rows[s], sem_g.at[s]
            ).wait()

        def fire_st(u, s):
            h, bb = u // NBB, u % NBB
            for db in range(D // 8):
                pltpu.make_async_copy(
                    tbuf[s].at[pl.ds(db * 8, 8)],
                    out_hbm.at[h, db, bb], sem_o.at[s],
                ).start()

        def wait_st(s):
            for db in range(D // 8):
                pltpu.make_async_copy(
                    tbuf[s].at[pl.ds(db * 8, 8)],
                    out_hbm.at[0, db, 0], sem_o.at[s],
                ).wait()

        lanes = lax.iota(jnp.int32, 16)

        def transpose(s):
            for d in range(D):
                dvec = jnp.full((16,), d, jnp.int32)
                for b0 in range(0, BB, 16):
                    v = plsc.load_gather(rows[s], [lanes + b0, dvec])
                    tbuf[s][d, pl.ds(b0, 16)] = v

        def step(u, s, first, last):
            wait_g(s)
            if not last:
                wait_idx(1 - s)
                fire_g(1 - s)
            if not first:
                wait_st(s)
            transpose(s)
            fire_st(u, s)

        fire_idx(u0, 0)
        fire_idx(u0 + 1, 1)
        wait_idx(0)
        fire_g(0)
        step(u0, 0, True, False)
        fire_idx(u0 + 2, 0)
        step(u0 + 1, 1, True, False)
        fire_idx(u0 + 3, 1)

        def body(i, carry):
            u = u0 + 2 * i
            step(u, 0, False, False)
            fire_idx(u + 2, 0)
            step(u + 1, 1, False, False)
            fire_idx(u + 3, 1)
            return carry

        lax.fori_loop(1, U_PER_W // 2 - 1, body, 0)

        step(u0 + U_PER_W - 2, 0, False, False)
        step(u0 + U_PER_W - 1, 1, False, True)
        wait_st(0)
        wait_st(1)

    return k(table, x_t)


def kernel(x, table):
    out5 = _embed(table, x.T)
    return out5.transpose(2, 4, 0, 1, 3).reshape(B, H, D)


# speedup vs baseline: 1.5758x; 1.1069x over previous
import functools

import jax
import jax.numpy as jnp
from jax import lax
from jax.experimental import pallas as pl
from jax.experimental.pallas import tpu as pltpu
from jax.experimental.pallas import tpu_sc as plsc

D = 32
NC = 2
NS = 16
NW = NC * NS
H = 50
B = 16384
BB = 128
NBB = B // BB
NBLK = NBB // NW
ROWS_W = NBLK * BB
U_PER_W = NBLK * H
DEPTH = 2


@jax.jit
def _embed(table, x):
    mesh = plsc.VectorSubcoreMesh(core_axis_name="c", subcore_axis_name="s")

    @functools.partial(
        pl.kernel,
        mesh=mesh,
        out_type=jax.ShapeDtypeStruct((H, D // 8, NBB, 8, BB), jnp.float32),
        scratch_types=[
            pltpu.VMEM((ROWS_W, H), jnp.int32),
            [pltpu.VMEM((BB,), jnp.int32) for _ in range(DEPTH)],
            [pltpu.VMEM((BB, D), jnp.float32) for _ in range(DEPTH)],
            [pltpu.VMEM((D, BB), jnp.float32) for _ in range(DEPTH)],
            pltpu.SemaphoreType.DMA,
            pltpu.SemaphoreType.DMA((DEPTH,)),
            pltpu.SemaphoreType.DMA((DEPTH,)),
        ],
        compiler_params=pltpu.CompilerParams(
            use_tc_tiling_on_sc=False, needs_layout_passes=False
        ),
    )
    def k(table_hbm, x_hbm, out_hbm, islab, ibuf, rows, tbuf, sem_b, sem_g, sem_o):
        wid = lax.axis_index("s") * NC + lax.axis_index("c")
        row0 = wid * ROWS_W
        bb0 = wid * NBLK

        pltpu.make_async_copy(
            x_hbm.at[pl.ds(row0, ROWS_W)], islab, sem_b
        ).start()

        lanes = lax.iota(jnp.int32, 16)

        def build_idx(u, s):
            h = u % H
            lb = (u // H) * BB
            hvec = jnp.full((16,), h, jnp.int32)
            for b0 in range(0, BB, 16):
                ibuf[s][pl.ds(b0, 16)] = plsc.load_gather(
                    islab, [lanes + (lb + b0), hvec]
                )

        def fire_g(s):
            pltpu.make_async_copy(
                table_hbm.at[ibuf[s]], rows[s], sem_g.at[s]
            ).start()

        def wait_g(s):
            pltpu.make_async_copy(
                table_hbm.at[ibuf[s]], rows[s], sem_g.at[s]
            ).wait()

        def fire_st(u, s):
            h = u % H
            bb = bb0 + u // H
            for db in range(D // 8):
                pltpu.make_async_copy(
                    tbuf[s].at[pl.ds(db * 8, 8)],
                    out_hbm.at[h, db, bb], sem_o.at[s],
                ).start()

        def wait_st(s):
            for db in range(D // 8):
                pltpu.make_async_copy(
                    tbuf[s].at[pl.ds(db * 8, 8)],
                    out_hbm.at[0, db, 0], sem_o.at[s],
                ).wait()

        def transpose(s):
            for d in range(D):
                dvec = jnp.full((16,), d, jnp.int32)
                for b0 in range(0, BB, 16):
                    v = plsc.load_gather(rows[s], [lanes + b0, dvec])
                    tbuf[s][d, pl.ds(b0, 16)] = v

        def step(u, s, first, last):
            wait_g(s)
            if not first:
                wait_st(s)
            transpose(s)
            fire_st(u, s)
            if not last:
                build_idx(u + DEPTH, s)
                fire_g(s)

        pltpu.make_async_copy(
            x_hbm.at[pl.ds(0, ROWS_W)], islab, sem_b
        ).wait()
        for j in range(DEPTH):
            build_idx(j, j)
            fire_g(j)
        for j in range(DEPTH):
            step(j, j, True, False)

        def body(i, carry):
            u = DEPTH * i
            for j in range(DEPTH):
                step(u + j, j, False, False)
            return carry

        lax.fori_loop(1, U_PER_W // DEPTH - 1, body, 0)

        for j in range(DEPTH):
            step(U_PER_W - DEPTH + j, j, False, True)
        for j in range(DEPTH):
            wait_st(j)

    return k(table, x)


def kernel(x, table):
    out5 = _embed(table, x)
    return out5.transpose(2, 4, 0, 1, 3).reshape(B, H, D)


# speedup vs baseline: 1.6224x; 1.0296x over previous
import functools

import jax
import jax.numpy as jnp
from jax import lax
from jax.experimental import pallas as pl
from jax.experimental.pallas import tpu as pltpu
from jax.experimental.pallas import tpu_sc as plsc

D = 32
NC = 2
NS = 16
NW = NC * NS
H = 50
B = 16384
BB = 128
NBB = B // BB
NBLK = NBB // NW
ROWS_W = NBLK * BB
U_PER_W = NBLK * H
RING = 8
NROUND = U_PER_W // RING


@jax.jit
def _embed(table, x):
    mesh = plsc.VectorSubcoreMesh(core_axis_name="c", subcore_axis_name="s")

    @functools.partial(
        pl.kernel,
        mesh=mesh,
        out_type=jax.ShapeDtypeStruct((H, D // 8, NBB, 8, BB), jnp.float32),
        scratch_types=[
            pltpu.VMEM((ROWS_W, H), jnp.int32),
            [pltpu.VMEM((BB,), jnp.int32) for _ in range(RING)],
            [pltpu.VMEM((BB, D), jnp.float32) for _ in range(RING)],
            [pltpu.VMEM((D, BB), jnp.float32) for _ in range(RING)],
            pltpu.SemaphoreType.DMA,
            pltpu.SemaphoreType.DMA((RING,)),
            pltpu.SemaphoreType.DMA((RING,)),
        ],
        compiler_params=pltpu.CompilerParams(
            use_tc_tiling_on_sc=False, needs_layout_passes=False
        ),
    )
    def k(table_hbm, x_hbm, out_hbm, islab, ibuf, rows, tbuf, sem_b, sem_g, sem_o):
        wid = lax.axis_index("s") * NC + lax.axis_index("c")
        row0 = wid * ROWS_W
        bb0 = wid * NBLK

        pltpu.make_async_copy(
            x_hbm.at[pl.ds(row0, ROWS_W)], islab, sem_b
        ).start()

        lanes = lax.iota(jnp.int32, 16)

        def build_idx(u, s):
            h = u % H
            lb = (u // H) * BB
            hvec = jnp.full((16,), h, jnp.int32)
            for b0 in range(0, BB, 16):
                ibuf[s][pl.ds(b0, 16)] = plsc.load_gather(
                    islab, [lanes + (lb + b0), hvec]
                )

        def fire_g(s):
            pltpu.make_async_copy(
                table_hbm.at[ibuf[s]], rows[s], sem_g.at[s]
            ).start()

        def wait_g(s):
            pltpu.make_async_copy(
                table_hbm.at[ibuf[s]], rows[s], sem_g.at[s]
            ).wait()

        def fire_st(u, s):
            h = u % H
            bb = bb0 + u // H
            for db in range(D // 8):
                pltpu.make_async_copy(
                    tbuf[s].at[pl.ds(db * 8, 8)],
                    out_hbm.at[h, db, bb], sem_o.at[s],
                ).start()

        def wait_st(s):
            for db in range(D // 8):
                pltpu.make_async_copy(
                    tbuf[s].at[pl.ds(db * 8, 8)],
                    out_hbm.at[0, db, 0], sem_o.at[s],
                ).wait()

        def transpose(s):
            def body(d, carry):
                dvec = jnp.full((16,), d, jnp.int32)
                for b0 in range(0, BB, 16):
                    v = plsc.load_gather(rows[s], [lanes + b0, dvec])
                    tbuf[s][d, pl.ds(b0, 16)] = v
                return carry

            lax.fori_loop(0, D, body, 0)

        def step(u, s, first, last):
            wait_g(s)
            if not first:
                wait_st(s)
            transpose(s)
            fire_st(u, s)
            if not last:
                build_idx(u + RING, s)
                fire_g(s)

        pltpu.make_async_copy(
            x_hbm.at[pl.ds(0, ROWS_W)], islab, sem_b
        ).wait()
        for s in range(RING):
            build_idx(s, s)
            fire_g(s)
        for s in range(RING):
            step(s, s, True, False)

        def body(i, carry):
            u = RING * i
            for s in range(RING):
                step(u + s, s, False, False)
            return carry

        lax.fori_loop(1, NROUND - 1, body, 0)

        for s in range(RING):
            step(U_PER_W - RING + s, s, False, True)
        for s in range(RING):
            wait_st(s)

    return k(table, x)


def kernel(x, table):
    out5 = _embed(table, x)
    return out5.transpose(2, 4, 0, 1, 3).reshape(B, H, D)


# speedup vs baseline: 2.5559x; 1.5754x over previous
import functools

import jax
import jax.numpy as jnp
from jax import lax
from jax.experimental import pallas as pl
from jax.experimental.pallas import tpu as pltpu
from jax.experimental.pallas import tpu_sc as plsc

D = 32
NC = 2
NS = 16
NW = NC * NS
H = 50
B = 16384
BB = 128
NBB = B // BB
NBLK = NBB // NW
ROWS_W = NBLK * BB
U_PER_W = NBLK * H
RING = 8
NROUND = U_PER_W // RING


@jax.jit
def _embed(table, x):
    mesh = plsc.VectorSubcoreMesh(core_axis_name="c", subcore_axis_name="s")

    @functools.partial(
        pl.kernel,
        mesh=mesh,
        out_type=jax.ShapeDtypeStruct((H, D // 8, NBB, 8, BB), jnp.float32),
        scratch_types=[
            pltpu.VMEM((ROWS_W, H), jnp.int32),
            [pltpu.VMEM((BB,), jnp.int32) for _ in range(RING)],
            [pltpu.VMEM((BB, D), jnp.float32) for _ in range(RING)],
            [pltpu.VMEM((D, BB + 1), jnp.float32) for _ in range(RING)],
            pltpu.SemaphoreType.DMA,
            pltpu.SemaphoreType.DMA((RING,)),
            pltpu.SemaphoreType.DMA((RING,)),
        ],
        compiler_params=pltpu.CompilerParams(
            use_tc_tiling_on_sc=False, needs_layout_passes=False
        ),
    )
    def k(table_hbm, x_hbm, out_hbm, islab, ibuf, rows, tbuf, sem_b, sem_g, sem_o):
        wid = lax.axis_index("s") * NC + lax.axis_index("c")
        row0 = wid * ROWS_W
        bb0 = wid * NBLK

        pltpu.make_async_copy(
            x_hbm.at[pl.ds(row0, ROWS_W)], islab, sem_b
        ).start()

        lanes = lax.iota(jnp.int32, 16)

        def build_idx(u, s):
            h = u % H
            lb = (u // H) * BB
            hvec = jnp.full((16,), h, jnp.int32)
            for b0 in range(0, BB, 16):
                ibuf[s][pl.ds(b0, 16)] = plsc.load_gather(
                    islab, [lanes + (lb + b0), hvec]
                )

        def fire_g(s):
            pltpu.make_async_copy(
                table_hbm.at[ibuf[s]], rows[s], sem_g.at[s]
            ).start()

        def wait_g(s):
            pltpu.make_async_copy(
                table_hbm.at[ibuf[s]], rows[s], sem_g.at[s]
            ).wait()

        def fire_st(u, s):
            h = u % H
            bb = bb0 + u // H
            for db in range(D // 8):
                pltpu.make_async_copy(
                    tbuf[s].at[pl.ds(db * 8, 8), pl.ds(0, BB)],
                    out_hbm.at[h, db, bb], sem_o.at[s],
                ).start()

        def wait_st(s):
            for db in range(D // 8):
                pltpu.make_async_copy(
                    tbuf[s].at[pl.ds(db * 8, 8), pl.ds(0, BB)],
                    out_hbm.at[0, db, 0], sem_o.at[s],
                ).wait()

        def transpose(s):
            def body(i, carry):
                for j in range(4):
                    b = 4 * i + j
                    bvec = jnp.full((16,), b, jnp.int32)
                    for d0 in range(0, D, 16):
                        v = rows[s][b, pl.ds(d0, 16)]
                        plsc.store_scatter(
                            tbuf[s], [lanes + d0, bvec], v
                        )
                return carry

            lax.fori_loop(0, BB // 4, body, 0)

        def step(u, s, first, last):
            wait_g(s)
            if not first:
                wait_st(s)
            transpose(s)
            fire_st(u, s)
            if not last:
                build_idx(u + RING, s)
                fire_g(s)

        pltpu.make_async_copy(
            x_hbm.at[pl.ds(0, ROWS_W)], islab, sem_b
        ).wait()
        for s in range(RING):
            build_idx(s, s)
            fire_g(s)
        for s in range(RING):
            step(s, s, True, False)

        def body(i, carry):
            u = RING * i
            for s in range(RING):
                step(u + s, s, False, False)
            return carry

        lax.fori_loop(1, NROUND - 1, body, 0)

        for s in range(RING):
            step(U_PER_W - RING + s, s, False, True)
        for s in range(RING):
            wait_st(s)

    return k(table, x)


def kernel(x, table):
    out5 = _embed(table, x)
    return out5.transpose(2, 4, 0, 1, 3).reshape(B, H, D)
